# baseline (speedup 1.0000x reference)
import jax
import jax.numpy as jnp
from jax import lax
from jax.experimental import pallas as pl
from jax.experimental.pallas import tpu as pltpu
from jax.experimental.pallas import tpu_sc as plsc

N = 50000
NH = 9
D = 128
F = NH * D
DW = D // 2

NW = 32
NPT = 1600
NPAD = NW * NPT
KG = 3
C = 80
NCH = NPT // C
G = C * KG
MMB = 2000

HI = -65536
RND = 0x8000


def _mm_body(x_ref, w_ref, o_ref):
    o_ref[...] = jnp.dot(x_ref[...], w_ref[...],
                         preferred_element_type=jnp.float32)


def _sc_body(adjc_hbm, pat_hbm, y_hbm, out_hbm,
             adjc_v0, adjc_v1, idx_v0, idx_v1, rows_v0, rows_v1,
             outb_v0, outb_v1, pat_v, sem0, sem1):
    cid = lax.axis_index("c")
    sid = lax.axis_index("s")
    wid = sid * 2 + cid
    base = wid * NPT
    pltpu.sync_copy(pat_hbm, pat_v)

    bufs = ((adjc_v0, idx_v0, rows_v0, outb_v0, sem0),
            (adjc_v1, idx_v1, rows_v1, outb_v1, sem1))

    def fire(ch, buf):
        adjc_v, idx_v, rows_v, _, sem = bufs[buf]
        row0 = base + ch * C
        pltpu.sync_copy(adjc_hbm.at[pl.ds(row0 * KG, G)], adjc_v)

        def idx_body(g, _):
            sl = pl.ds(g * 16, 16)
            idx_v[sl] = adjc_v[sl] * KG + pat_v[sl]
            return 0
        lax.fori_loop(0, G // 16, idx_body, 0, unroll=4)
        pltpu.async_copy(y_hbm.at[idx_v], rows_v, sem)

    def consume(ch, buf):
        _, idx_v, rows_v, outb_v, sem = bufs[buf]
        pltpu.make_async_copy(y_hbm.at[idx_v], rows_v, sem).wait()

        def sum_body(i, _):
            r0 = i * KG
            for j in range(D // 16):
                js = pl.ds(j * 16, 16)
                acc = rows_v[r0, js]
                for k in range(1, KG):
                    acc = acc + rows_v[r0 + k, js]
                outb_v[i, js] = acc
            return 0
        lax.fori_loop(0, C, sum_body, 0)
        pltpu.sync_copy(outb_v, out_hbm.at[pl.ds(base + ch * C, C)])

    fire(0, 0)

    def pair_body(p, _):
        ch0 = p * 2
        fire(ch0 + 1, 1)
        consume(ch0, 0)

        @pl.when(p < NCH // 2 - 1)
        def _():
            fire(ch0 + 2, 0)
        consume(ch0 + 1, 1)
        return 0

    lax.fori_loop(0, NCH // 2, pair_body, 0)


def kernel(x, adjc, W, b):
    x2 = x.reshape(N, D)
    W2 = W.reshape(NH, D, D).transpose(1, 0, 2).reshape(D, F)

    mesh = plsc.VectorSubcoreMesh(core_axis_name="c", subcore_axis_name="s")
    sc_call = pl.kernel(
        _sc_body,
        out_type=jax.ShapeDtypeStruct((NPAD, D), jnp.float32),
        mesh=mesh,
        scratch_types=[
            pltpu.VMEM((G,), jnp.int32),
            pltpu.VMEM((G,), jnp.int32),
            pltpu.VMEM((G,), jnp.int32),
            pltpu.VMEM((G,), jnp.int32),
            pltpu.VMEM((G, D), jnp.float32),
            pltpu.VMEM((G, D), jnp.float32),
            pltpu.VMEM((C, D), jnp.float32),
            pltpu.VMEM((C, D), jnp.float32),
            pltpu.VMEM((G,), jnp.int32),
            pltpu.SemaphoreType.DMA,
            pltpu.SemaphoreType.DMA,
        ],
    )
    pat = jnp.tile(jnp.arange(KG, dtype=jnp.int32), C)

    FG = KG * D
    parts = []
    for g in range(NH // KG):
        Wg = W2[:, g * FG:(g + 1) * FG]
        Yg = pl.pallas_call(
            _mm_body,
            grid=(N // MMB,),
            in_specs=[pl.BlockSpec((MMB, D), lambda i: (i, 0)),
                      pl.BlockSpec((D, FG), lambda i: (0, 0))],
            out_specs=pl.BlockSpec((MMB, FG), lambda i: (i, 0)),
            out_shape=jax.ShapeDtypeStruct((N, FG), jnp.float32),
        )(x2, Wg)
        adjg = jnp.pad(adjc[:, g * KG:(g + 1) * KG].reshape(-1),
                       (0, (NPAD - N) * KG))
        parts.append(sc_call(adjg, pat, Yg.reshape(N * KG, D)))

    out = parts[0][:N] + parts[1][:N] + parts[2][:N] + b
    return out.reshape(1, 1, N, 1, D)


# speedup vs baseline: 1.3575x; 1.3575x over previous
import jax
import jax.numpy as jnp
from jax import lax
from jax.experimental import pallas as pl
from jax.experimental.pallas import tpu as pltpu
from jax.experimental.pallas import tpu_sc as plsc

N = 50000
NH = 9
D = 128
F = NH * D
DW = D // 2

NW = 32
NPT = 1600
NPAD = NW * NPT
C = 32
NCH = NPT // C
G = C * NH
MMB = 2000

HI = -65536
RND = 0x8000


def _mm_body(x_ref, w_ref, o_ref):
    o_ref[...] = jnp.dot(x_ref[...].astype(jnp.bfloat16), w_ref[...],
                         preferred_element_type=jnp.float32)


def _sc_body(adjc_hbm, pat_hbm, y_hbm, b_hbm, out_hbm,
             adjc_a, idx_a, rows_v0, rows_v1, outb_v0, outb_v1, b_v, pat_v,
             gsem0, gsem1, osem0, osem1):
    cid = lax.axis_index("c")
    sid = lax.axis_index("s")
    wid = sid * 2 + cid
    base = wid * NPT
    pltpu.sync_copy(b_hbm, b_v)
    pltpu.sync_copy(pat_hbm, pat_v)
    pltpu.sync_copy(adjc_hbm.at[pl.ds(base * NH, NPT * NH)], adjc_a)

    def idx_body(g, _):
        sl = pl.ds(g * 16, 16)
        ps = pl.ds(lax.rem(g, NH) * 16, 16)
        idx_a[sl] = adjc_a[sl] * NH + pat_v[ps]
        return 0
    lax.fori_loop(0, (NPT * NH) // 16, idx_body, 0, unroll=8)

    bufs = ((rows_v0, outb_v0, gsem0, osem0),
            (rows_v1, outb_v1, gsem1, osem1))

    def fire(ch, buf):
        rows_v, _, gsem, _ = bufs[buf]
        pltpu.async_copy(y_hbm.at[idx_a.at[pl.ds(ch * G, G)]], rows_v, gsem)

    def consume(ch, p, buf):
        rows_v, outb_v, gsem, osem = bufs[buf]
        pltpu.make_async_copy(y_hbm.at[idx_a.at[pl.ds(ch * G, G)]],
                              rows_v, gsem).wait()

        @pl.when(p > 0)
        def _():
            pltpu.make_async_copy(outb_v, out_hbm.at[pl.ds(base, C)],
                                  osem).wait()

        def sum_body(i, _):
            r0 = i * NH
            for j in range(D // 16):
                js = pl.ds(j * 16, 16)
                acc = b_v[js]
                for k in range(NH):
                    acc = acc + rows_v[r0 + k, js]
                outb_v[i, js] = acc
            return 0
        lax.fori_loop(0, C, sum_body, 0)
        pltpu.async_copy(outb_v, out_hbm.at[pl.ds(base + ch * C, C)], osem)

        @pl.when(ch + 2 < NCH)
        def _():
            fire(ch + 2, buf)

    fire(0, 0)
    fire(1, 1)

    def pair_body(p, _):
        consume(p * 2, p, 0)
        consume(p * 2 + 1, p, 1)
        return 0

    lax.fori_loop(0, NCH // 2, pair_body, 0)
    pltpu.make_async_copy(outb_v0, out_hbm.at[pl.ds(base, C)], osem0).wait()
    pltpu.make_async_copy(outb_v1, out_hbm.at[pl.ds(base, C)], osem1).wait()


def kernel(x, adjc, W, b):
    x2 = x.reshape(N, D)
    W2 = W.reshape(NH, D, D).transpose(1, 0, 2).reshape(D, F)
    W2 = W2.astype(jnp.bfloat16)

    Y = pl.pallas_call(
        _mm_body,
        grid=(N // MMB,),
        in_specs=[pl.BlockSpec((MMB, D), lambda i: (i, 0)),
                  pl.BlockSpec((D, F), lambda i: (0, 0))],
        out_specs=pl.BlockSpec((MMB, F), lambda i: (i, 0)),
        out_shape=jax.ShapeDtypeStruct((N, F), jnp.float32),
    )(x2, W2)
    Yr = Y.reshape(N * NH, D)

    adjc_flat = jnp.pad(adjc.reshape(-1), (0, (NPAD - N) * NH))
    pat = (jnp.arange(NH * 16, dtype=jnp.int32)) % NH


    mesh = plsc.VectorSubcoreMesh(core_axis_name="c", subcore_axis_name="s")
    out_sc = pl.kernel(
        _sc_body,
        out_type=jax.ShapeDtypeStruct((NPAD, D), jnp.float32),
        mesh=mesh,
        scratch_types=[
            pltpu.VMEM((NPT * NH,), jnp.int32),
            pltpu.VMEM((NPT * NH,), jnp.int32),
            pltpu.VMEM((G, D), jnp.float32),
            pltpu.VMEM((G, D), jnp.float32),
            pltpu.VMEM((C, D), jnp.float32),
            pltpu.VMEM((C, D), jnp.float32),
            pltpu.VMEM((D,), jnp.float32),
            pltpu.VMEM((NH * 16,), jnp.int32),
            pltpu.SemaphoreType.DMA,
            pltpu.SemaphoreType.DMA,
            pltpu.SemaphoreType.DMA,
            pltpu.SemaphoreType.DMA,
        ],
    )(adjc_flat, pat, Yr, b)

    return out_sc[:N].reshape(1, 1, N, 1, D)
